# baseline (speedup 1.0000x reference)
import dataclasses
import functools

import jax
import jax.numpy as jnp
from jax import lax
from jax.experimental import pallas as pl
from jax.experimental.pallas import tpu as pltpu
from jax.experimental.pallas import tpu_sc as plsc

_NC = 2
_NS = 16
_LANES = 16
_BLK = 128


def _edge_proj(edge_attr, We_l):
    E, DE = edge_attr.shape
    H = We_l.shape[1]
    BE = 2560
    assert E % BE == 0

    def body(a_ref, w_ref, o_ref):
        o_ref[...] = lax.dot_general(
            a_ref[...], w_ref[...], (((1,), (0,)), ((), ())),
            preferred_element_type=jnp.float32)

    edge_attr = edge_attr.astype(jnp.bfloat16)
    We_l = We_l.astype(jnp.bfloat16)

    return pl.pallas_call(
        body,
        grid=(E // BE,),
        in_specs=[pl.BlockSpec((BE, DE), lambda i: (i, 0)),
                  pl.BlockSpec((DE, H), lambda i: (0, 0))],
        out_specs=pl.BlockSpec((BE, H), lambda i: (i, 0)),
        out_shape=jax.ShapeDtypeStruct((E, H), jnp.float32),
    )(edge_attr, We_l)


@functools.cache
def _make_edge_agg(N, E, H):
    NW = _NC * _NS
    BLK = 64
    n_blocks = E // BLK
    assert n_blocks * BLK == E
    bpt = (n_blocks + NW - 1) // NW
    zfull = N // BLK
    zrem = N - zfull * BLK
    zch_per_tile = (zfull + _NS) // _NS
    mesh = plsc.VectorSubcoreMesh(core_axis_name="c", subcore_axis_name="s")

    @functools.partial(
        pl.kernel,
        mesh=mesh,
        out_type=jax.ShapeDtypeStruct((_NC, N, H), jnp.float32),
        scratch_types=[
            pltpu.VMEM((3, BLK), jnp.int32),
            pltpu.VMEM((3, BLK), jnp.int32),
            pltpu.VMEM((3, BLK, H), jnp.float32),
            pltpu.VMEM((3, BLK, H), jnp.float32),
            pltpu.VMEM_SHARED((N, H), jnp.float32),
            pltpu.SemaphoreType.DMA,
            pltpu.SemaphoreType.DMA,
            pltpu.SemaphoreType.DMA,
            pltpu.SemaphoreType.DMA,
            pltpu.SemaphoreType.DMA,
            pltpu.SemaphoreType.DMA,
            pltpu.SemaphoreType.DMA,
            pltpu.SemaphoreType.DMA,
            pltpu.SemaphoreType.DMA,
        ],
    )
    def edge_agg(h_hbm, e_hbm, src_hbm, dst_hbm, out_hbm,
                 src3, dst3, hrows3, erows3, agg_sh,
                 si0, si1, si2, sg0, sg1, sg2, se0, se1, se2):
        c = lax.axis_index("c")
        s = lax.axis_index("s")
        wid = c * _NS + s
        sem_i = (si0, si1, si2)
        sem_g = (sg0, sg1, sg2)
        sem_e = (se0, se1, se2)

        zvec = jnp.zeros((_LANES,), jnp.float32)

        @pl.loop(0, BLK)
        def _(i):
            for j in range(H // _LANES):
                hrows3[0, i, pl.ds(j * _LANES, _LANES)] = zvec

        @pl.loop(0, zch_per_tile)
        def _(k):
            ch = k * _NS + s

            @pl.when(ch < zfull)
            def _():
                pltpu.sync_copy(hrows3.at[0], agg_sh.at[pl.ds(ch * BLK, BLK)])

            if zrem:
                @pl.when(ch == zfull)
                def _():
                    pltpu.sync_copy(hrows3.at[0].at[pl.ds(0, zrem)],
                                    agg_sh.at[pl.ds(zfull * BLK, zrem)])

        plsc.subcore_barrier()

        def gid(i):
            return i * NW + wid

        def ok(i):
            return gid(i) < n_blocks

        def start_idx(slot, i):
            base = gid(i) * BLK
            pltpu.async_copy(src_hbm.at[pl.ds(base, BLK)], src3.at[slot],
                             sem_i[slot])
            pltpu.async_copy(dst_hbm.at[pl.ds(base, BLK)], dst3.at[slot],
                             sem_i[slot])

        def wait_idx(slot):
            pltpu.make_async_copy(src_hbm.at[pl.ds(0, BLK)], src3.at[slot],
                                  sem_i[slot]).wait()
            pltpu.make_async_copy(dst_hbm.at[pl.ds(0, BLK)], dst3.at[slot],
                                  sem_i[slot]).wait()

        def start_data(slot, i):
            base = gid(i) * BLK
            pltpu.async_copy(h_hbm.at[src3.at[slot]], hrows3.at[slot],
                             sem_g[slot])
            pltpu.async_copy(e_hbm.at[pl.ds(base, BLK)], erows3.at[slot],
                             sem_e[slot])

        def wait_data(slot):
            pltpu.make_async_copy(h_hbm.at[src3.at[slot]], hrows3.at[slot],
                                  sem_g[slot]).wait()
            pltpu.make_async_copy(e_hbm.at[pl.ds(0, BLK)], erows3.at[slot],
                                  sem_e[slot]).wait()

        def step(i, u):
            slot = u
            nx2 = (u + 2) % 3

            @pl.when(ok(i))
            def _():
                wait_data(slot)

            @pl.when(ok(i + 2))
            def _():
                wait_idx(nx2)
                start_data(nx2, i + 2)

            @pl.when(ok(i))
            def _():
                @pl.loop(0, BLK)
                def _(k):
                    for j in range(H // _LANES):
                        sl = pl.ds(j * _LANES, _LANES)
                        hv = hrows3[slot, k, sl]
                        ev = erows3[slot, k, sl]
                        hrows3[slot, k, sl] = jnp.maximum(hv + ev, 0.0)

                pltpu.sync_copy(hrows3.at[slot], agg_sh.at[dst3.at[slot]],
                                add=True)

                @pl.when(ok(i + 3))
                def _():
                    start_idx(slot, i + 3)

        start_idx(0, 0)
        wait_idx(0)
        start_data(0, 0)
        start_idx(1, 1)
        wait_idx(1)
        start_data(1, 1)
        start_idx(2, 2)

        step(0, 0)

        n_rest = bpt - 1
        assert n_rest % 3 == 0, n_rest

        @pl.loop(0, n_rest // 3)
        def _(k):
            i0 = k * 3 + 1
            for t in range(3):
                step(i0 + t, (t + 1) % 3)

        plsc.subcore_barrier()

        @pl.loop(0, zch_per_tile)
        def _(k):
            ch = k * _NS + s

            @pl.when(ch < zfull)
            def _():
                pltpu.sync_copy(agg_sh.at[pl.ds(ch * BLK, BLK)],
                                out_hbm.at[c].at[pl.ds(ch * BLK, BLK)])

            if zrem:
                @pl.when(ch == zfull)
                def _():
                    pltpu.sync_copy(agg_sh.at[pl.ds(zfull * BLK, zrem)],
                                    out_hbm.at[c].at[pl.ds(zfull * BLK, zrem)])

    return edge_agg


def _node_update(h_in, parts, W1l, b1l, W2l, b2l, gammal, betal, relu_out):
    N, H = h_in.shape

    def body(h_ref, p_ref, w1, b1, w2, b2, ga, be, o_ref):
        z = h_ref[...] + p_ref[0] + p_ref[1]
        u = lax.dot_general(z, w1[...], (((1,), (0,)), ((), ())),
                            preferred_element_type=jnp.float32) + b1[...]
        u = jnp.maximum(u, 0.0)
        v = lax.dot_general(u, w2[...], (((1,), (0,)), ((), ())),
                            preferred_element_type=jnp.float32) + b2[...]
        mu = jnp.mean(v, axis=0, keepdims=True)
        var = jnp.mean((v - mu) * (v - mu), axis=0, keepdims=True)
        zn = (v - mu) * lax.rsqrt(var + 1e-5) * ga[...] + be[...]
        if relu_out:
            zn = jnp.maximum(zn, 0.0)
        o_ref[...] = zn + h_ref[...]

    return pl.pallas_call(
        body,
        out_shape=jax.ShapeDtypeStruct((N, H), jnp.float32),
    )(h_in, parts, W1l, b1l, W2l, b2l, gammal, betal)


@functools.cache
def _make_readout(N, H, G):
    NW = _NC * _NS
    GPT = G // NW
    CH = N // _LANES
    assert CH * _LANES == N
    W = 64
    mesh = plsc.VectorSubcoreMesh(core_axis_name="c", subcore_axis_name="s")
    cp = pltpu.CompilerParams()
    if "needs_layout_passes" in pltpu.CompilerParams.__dataclass_fields__:
        cp = dataclasses.replace(cp, needs_layout_passes=False)

    @functools.partial(
        pl.kernel,
        mesh=mesh,
        compiler_params=cp,
        out_type=jax.ShapeDtypeStruct((NW, GPT, H), jnp.float32),
        scratch_types=[
            pltpu.VMEM((N,), jnp.int32),
            pltpu.VMEM((W, H), jnp.float32),
            pltpu.VMEM((GPT, H), jnp.float32),
            pltpu.SemaphoreType.DMA,
        ],
    )
    def readout(h_hbm, b_hbm, out_hbm, bv, rows, acc, sem):
        c = lax.axis_index("c")
        s = lax.axis_index("s")
        wid = c * _NS + s
        g0 = wid * GPT

        pltpu.async_copy(b_hbm, bv, sem).wait()

        zero = jnp.zeros((_LANES,), jnp.int32)
        one = jnp.ones((_LANES,), jnp.int32)

        @pl.loop(0, CH, init_carry=(zero,) * (GPT + 1))
        def counts(k, carry):
            ch = bv[pl.ds(k * _LANES, _LANES)]
            return tuple(
                carry[q] + jnp.where(ch < g0 + q, one, zero)
                for q in range(GPT + 1))

        cnts = [jnp.sum(v) for v in counts]

        ninf = jnp.full((_LANES,), -jnp.inf, jnp.float32)
        for q in range(GPT):
            for j in range(H // _LANES):
                acc[q, pl.ds(j * _LANES, _LANES)] = ninf

        for q in range(GPT):
            start = cnts[q]
            end = cnts[q + 1]
            start8 = start - lax.rem(start, 8)
            nwin = jnp.maximum((end - start8 + W - 1) // W, 0)

            @pl.loop(0, nwin)
            def _(k):
                w0 = pl.multiple_of(jnp.minimum(start8 + k * W, N - W), 8)
                pltpu.async_copy(h_hbm.at[pl.ds(w0, W)], rows, sem).wait()

                @pl.loop(0, W)
                def _(r):
                    row = w0 + r

                    @pl.when(jnp.logical_and(row >= start, row < end))
                    def _():
                        for j in range(H // _LANES):
                            sl = pl.ds(j * _LANES, _LANES)
                            acc[q, sl] = jnp.maximum(acc[q, sl], rows[r, sl])

        pltpu.sync_copy(acc, out_hbm.at[wid])

    return readout


def kernel(x, edge_index, edge_attr, batch, W1, b1, W2, b2, We, gamma, beta):
    N, H = x.shape
    E = edge_index.shape[1]
    L = W1.shape[0]
    G = 128

    src = edge_index[0]
    dst = edge_index[1]
    edge_agg = _make_edge_agg(N, E, H)

    es = [_edge_proj(edge_attr, We[l]) for l in range(L)]
    h = x
    for l in range(L):
        parts = edge_agg(h, es[l], src, dst)
        h = _node_update(h, parts,
                         W1[l], b1[l].reshape(1, -1),
                         W2[l], b2[l].reshape(1, -1),
                         gamma[l].reshape(1, -1), beta[l].reshape(1, -1),
                         relu_out=(l < L - 1))
    h_rep = _make_readout(N, H, G)(h, batch).reshape(G, H)
    return h_rep, h
